# Initial kernel scaffold; baseline (speedup 1.0000x reference)
import jax
import jax.numpy as jnp
from jax.experimental import pallas as pl


def kernel(node_atom, node_tag, pos, edge_index, batch, atom_emb, tag_emb, Wd1, Wd2, Wd3s, Wd3v, Wq, Wk, Wvs, Wvv, We, Wos, Wov, Wf1, Wf2, Wg0, Wg1, Wg2, Wh1, Wh2):
    raise NotImplementedError("write your pallas kernel here")


# speedup vs baseline: 1.1647x; 1.1647x over previous
import functools
import math

import jax
import jax.numpy as jnp
from jax import lax
from jax.experimental import pallas as pl
from jax.experimental.pallas import tpu as pltpu

N = 10000
E = 160000
D0 = 256
D1 = 128
L = 2
H = 8
DH = 32
NB = 128
NG = 128
AVG_DEG = 23.395238876342773
AVG_NODES = 77.81317
MAX_R = 6.0

BE = 2000


def _ln(x):
    m = x.mean(-1, keepdims=True)
    v = ((x - m) ** 2).mean(-1, keepdims=True)
    return (x - m) / jnp.sqrt(v + 1e-5)


def _edge_geom_kernel(ps_ref, pd_ref, Wd1_ref, Wd2_ref, We2_ref, hh_ref, eb_ref):
    ps = ps_ref[...]
    pd = pd_ref[...]
    rel = ps - pd
    d2 = (rel * rel).sum(axis=1, keepdims=True) + 1e-12
    dist = jnp.sqrt(d2)
    inv = 1.0 / dist
    shx = rel[:, 0:1] * inv
    shy = rel[:, 1:2] * inv
    shz = rel[:, 2:3] * inv
    step = MAX_R / (NB - 1)
    width = MAX_R / NB
    centers = lax.broadcasted_iota(jnp.int32, (1, NB), 1).astype(jnp.float32) * step
    t = (dist - centers) * (1.0 / width)
    rbf = jnp.exp(-0.5 * t * t)
    h = rbf @ Wd1_ref[...]
    h = h * jax.nn.sigmoid(h)
    h = h @ Wd2_ref[...]
    h = h * jax.nn.sigmoid(h)
    hh_ref[:, 0:64] = h
    hh_ref[:, 64:128] = h * shx
    hh_ref[:, 128:192] = h * shy
    hh_ref[:, 192:256] = h * shz
    eb_ref[...] = rbf @ We2_ref[...]


def _edge_geom(pos_src, pos_dst, Wd1, Wd2, We2):
    grid = (E // BE,)
    return pl.pallas_call(
        _edge_geom_kernel,
        grid=grid,
        in_specs=[
            pl.BlockSpec((BE, 3), lambda i: (i, 0)),
            pl.BlockSpec((BE, 3), lambda i: (i, 0)),
            pl.BlockSpec((NB, 64), lambda i: (0, 0)),
            pl.BlockSpec((64, 64), lambda i: (0, 0)),
            pl.BlockSpec((NB, 2 * H), lambda i: (0, 0)),
        ],
        out_specs=[
            pl.BlockSpec((BE, 256), lambda i: (i, 0)),
            pl.BlockSpec((BE, 2 * H), lambda i: (i, 0)),
        ],
        out_shape=[
            jax.ShapeDtypeStruct((E, 256), jnp.float32),
            jax.ShapeDtypeStruct((E, 2 * H), jnp.float32),
        ],
    )(pos_src, pos_dst, Wd1, Wd2, We2)


def kernel(node_atom, node_tag, pos, edge_index, batch, atom_emb, tag_emb,
           Wd1, Wd2, Wd3s, Wd3v, Wq, Wk, Wvs, Wvv, We, Wos, Wov,
           Wf1, Wf2, Wg0, Wg1, Wg2, Wh1, Wh2):
    src = edge_index[0]
    dst = edge_index[1]
    n = pos.shape[0]

    s = atom_emb[node_atom] + tag_emb[node_tag]
    v = jnp.zeros((n, D1, 3), dtype=pos.dtype)

    pos_src = pos[src]
    pos_dst = pos[dst]
    We2 = jnp.concatenate([We[0], We[1]], axis=1)
    hh, eb = _edge_geom(pos_src, pos_dst, Wd1, Wd2, We2)

    A = jax.ops.segment_sum(hh, dst, num_segments=n)
    inv_sqrt_deg = 1.0 / math.sqrt(AVG_DEG)
    s = s + (A[:, 0:64] @ Wd3s) * inv_sqrt_deg
    deg_v = jnp.stack(
        [A[:, 64 + 64 * d:128 + 64 * d] @ Wd3v for d in range(3)], axis=-1)
    v = v + deg_v * inv_sqrt_deg

    for l in range(L):
        s_in = _ln(s)
        q = (s_in @ Wq[l])[dst].reshape(-1, H, DH)
        k = (s_in @ Wk[l])[src].reshape(-1, H, DH)
        alpha = (q * k).sum(-1) / math.sqrt(DH) + eb[:, l * H:(l + 1) * H]
        ex = jnp.exp(alpha)
        den = jax.ops.segment_sum(ex, dst, num_segments=n)
        attn = ex / den[dst]
        vs = (s_in @ Wvs[l])[src].reshape(-1, H, DH) * attn[:, :, None]
        msg_s = jax.ops.segment_sum(vs.reshape(-1, H * DH), dst, num_segments=n)
        vv = jnp.einsum('ncd,cm->nmd', v, Wvv[l])[src]
        vv = vv.reshape(-1, H, D1 // H, 3) * attn[:, :, None, None]
        msg_v = jax.ops.segment_sum(vv.reshape(-1, D1, 3), dst, num_segments=n)
        s = s + msg_s @ Wos[l]
        v = v + jnp.einsum('ncd,cm->nmd', msg_v, Wov[l])
        s_n = _ln(s)
        s = s + jax.nn.silu(s_n @ Wf1[l]) @ Wf2[l]
        gate = jax.nn.sigmoid(s_n @ Wg0[l])
        vmid = jnp.einsum('ncd,cm->nmd', v, Wg1[l]) * gate[:, :, None]
        v = v + jnp.einsum('nmd,mc->ncd', vmid, Wg2[l])

    sf = _ln(s)
    e = jax.nn.silu(sf @ Wh1) @ Wh2
    energy = jax.ops.segment_sum(e, batch, num_segments=NG) / AVG_NODES
    return energy
